# baseline (speedup 1.0000x reference)
import jax
import jax.numpy as jnp
from jax import lax
from jax.experimental import pallas as pl
from jax.experimental.pallas import tpu as pltpu
from jax.experimental.pallas import tpu_sc as plsc

N_NODES = 10000
N_PAD = 10240
N_EDGES = 320000
C_IN = 128
CH = C_IN // 2
NC, NS, L = 2, 16, 16
E_TILE = N_EDGES // NS
K = 80
STEPS = E_TILE // K
NBUF = 4
CHUNK = 80
NCHUNK = N_NODES // CHUNK
RBLK = 1000
CBLK = 1024
PSHIFT = 14


def _prep_body(x_ref, w_ref, att_ref, h_ref, s_ref):
    h = lax.dot_general(x_ref[...], w_ref[...], (((1,), (1,)), ((), ())),
                        preferred_element_type=jnp.float32)
    h_ref[0] = h[:, :CH]
    h_ref[1] = h[:, CH:]
    s_ref[...] = lax.dot_general(h, att_ref[...], (((1,), (1,)), ((), ())),
                                 preferred_element_type=jnp.float32)


def _edge_body(hflat, sN, pk3d, acc_out, den_out,
               s_v, pk_v,
               rb0, rb1, rb2, rb3, cb0, cb1, cb2, cb3,
               av0, av1, av2, av3, den_v,
               gb0, gb1, gb2, gb3, accS,
               gs0, gs1, gs2, gs3, ss0, ss1, ss2, ss3):
    rowbufs = [rb0, rb1, rb2, rb3]
    colbufs = [cb0, cb1, cb2, cb3]
    alpha_vs = [av0, av1, av2, av3]
    gbufs = [gb0, gb1, gb2, gb3]
    gsems = [gs0, gs1, gs2, gs3]
    ssems = [ss0, ss1, ss2, ss3]

    c = lax.axis_index("c")
    s = lax.axis_index("s")

    pltpu.sync_copy(pk3d.at[s], pk_v)
    pltpu.sync_copy(sN, s_v)

    def drow(r, _):
        den_v[pl.ds(r * L, L)] = jnp.zeros((L,), jnp.float32)
        return 0
    lax.fori_loop(0, N_NODES // L, drow, 0)

    def zrow(r, _):
        for v in range(CH // L):
            gb0[r, pl.ds(v * L, L)] = jnp.zeros((L,), jnp.float32)
        return 0
    lax.fori_loop(0, CHUNK, zrow, 0)
    for i in range((NCHUNK + NS - 1) // NS):
        cid = i * NS + s

        @pl.when(cid < NCHUNK)
        def _():
            pltpu.sync_copy(gb0, accS.at[pl.ds(cid * CHUNK, CHUNK)])
    plsc.subcore_barrier()

    hoff = c * N_NODES

    def prep(j, b):
        for k in range(K // L):
            pk = pk_v[j, pl.ds(k * L, L)]
            rv = lax.shift_right_logical(pk, PSHIFT)
            cv = lax.bitwise_and(pk, (1 << PSHIFT) - 1)
            rowbufs[b][pl.ds(k * L, L)] = rv + hoff
            colbufs[b][pl.ds(k * L, L)] = cv
            t = (plsc.load_gather(s_v, [rv * 2])
                 + plsc.load_gather(s_v, [cv * 2 + 1]))
            t = jnp.maximum(t, t * 0.2)
            a = jnp.exp(t)
            alpha_vs[b][pl.ds(k * L, L)] = a
            plsc.addupdate_scatter(den_v, [cv], a)
        pltpu.async_copy(hflat.at[rowbufs[b]], gbufs[b], gsems[b])

    def wait_scatter(b):
        pltpu.make_async_copy(gbufs[b], accS.at[colbufs[b]],
                              ssems[b]).wait()

    def consume(j, b):
        pltpu.make_async_copy(hflat.at[rowbufs[b]], gbufs[b],
                              gsems[b]).wait()
        gb = gbufs[b]
        av = alpha_vs[b]

        def srow16(r16, _):
            r0 = r16 * L
            avv = av[pl.ds(r0, L)]
            for i in range(L):
                a = avv[i]
                for v in range(CH // L):
                    sl = pl.ds(v * L, L)
                    gb[r0 + i, sl] = gb[r0 + i, sl] * a
            return 0
        lax.fori_loop(0, K // L, srow16, 0)
        pltpu.async_copy(gbufs[b], accS.at[colbufs[b]], ssems[b],
                         add=True)

    prep(0, 0)

    def quad(i, _):
        for b in range(NBUF):
            j = NBUF * i + b
            nb = (b + 1) % NBUF

            @pl.when(jnp.logical_and(j + 1 < STEPS, j >= NBUF - 1))
            def _():
                wait_scatter(nb)

            @pl.when(j + 1 < STEPS)
            def _():
                prep(j + 1, nb)

            @pl.when(j < STEPS)
            def _():
                consume(j, b)
        return 0

    lax.fori_loop(0, (STEPS + NBUF - 1) // NBUF, quad, 0)
    for jt in range(STEPS - NBUF, STEPS):
        wait_scatter(jt % NBUF)

    @pl.when(c == 0)
    def _():
        pltpu.sync_copy(den_v, den_out.at[s, pl.ds(0, N_NODES)])
    plsc.subcore_barrier()
    for i in range((NCHUNK + NS - 1) // NS):
        cid = i * NS + s

        @pl.when(cid < NCHUNK)
        def _():
            pltpu.sync_copy(accS.at[pl.ds(cid * CHUNK, CHUNK)],
                            acc_out.at[c, pl.ds(cid * CHUNK, CHUNK)])


def _combine_body(a0_ref, a1_ref, den_ref, o_ref):
    den = jnp.sum(den_ref[...], axis=0) + 1e-16
    o_ref[...] = jnp.concatenate([a0_ref[...], a1_ref[...]],
                                 axis=1) / den[:, None]


def kernel(x, edge_index, W, att):
    x = x.astype(jnp.float32)
    ei = edge_index.astype(jnp.int32)
    packed = (ei[0] * (1 << PSHIFT) + ei[1]).reshape(NS, STEPS, K)
    att_r = att.reshape(2, C_IN)

    hsplit, sN = pl.pallas_call(
        _prep_body,
        grid=(N_NODES // RBLK,),
        in_specs=[
            pl.BlockSpec((RBLK, C_IN), lambda i: (i, 0)),
            pl.BlockSpec((C_IN, C_IN), lambda i: (0, 0)),
            pl.BlockSpec((2, C_IN), lambda i: (0, 0)),
        ],
        out_specs=[
            pl.BlockSpec((2, RBLK, CH), lambda i: (0, i, 0)),
            pl.BlockSpec((RBLK, 2), lambda i: (i, 0)),
        ],
        out_shape=[
            jax.ShapeDtypeStruct((2, N_NODES, CH), jnp.float32),
            jax.ShapeDtypeStruct((N_NODES, 2), jnp.float32),
        ],
    )(x, W, att_r)

    mesh = plsc.VectorSubcoreMesh(core_axis_name="c", subcore_axis_name="s",
                                  num_cores=NC, num_subcores=NS)
    acc, den = pl.kernel(
        _edge_body,
        out_type=[
            jax.ShapeDtypeStruct((NC, N_PAD, CH), jnp.float32),
            jax.ShapeDtypeStruct((NS, N_PAD), jnp.float32),
        ],
        mesh=mesh,
        compiler_params=pltpu.CompilerParams(needs_layout_passes=False,
                                             use_tc_tiling_on_sc=False),
        scratch_types=[
            pltpu.VMEM((2 * N_NODES,), jnp.float32),
            pltpu.VMEM((STEPS, K), jnp.int32),
            *([pltpu.VMEM((K,), jnp.int32)] * (2 * NBUF)),
            *([pltpu.VMEM((K + L,), jnp.float32)] * NBUF),
            pltpu.VMEM((N_NODES,), jnp.float32),
            *([pltpu.VMEM((K, CH), jnp.float32)] * NBUF),
            pltpu.VMEM_SHARED((N_NODES, CH), jnp.float32),
            *([pltpu.SemaphoreType.DMA] * (2 * NBUF)),
        ],
    )(hsplit.reshape(2 * N_NODES, CH), sN.reshape(2 * N_NODES), packed)

    out = pl.pallas_call(
        _combine_body,
        grid=(N_PAD // CBLK,),
        in_specs=[
            pl.BlockSpec((CBLK, CH), lambda i: (i, 0)),
            pl.BlockSpec((CBLK, CH), lambda i: (i, 0)),
            pl.BlockSpec((NS, CBLK), lambda i: (0, i)),
        ],
        out_specs=pl.BlockSpec((CBLK, C_IN), lambda i: (i, 0)),
        out_shape=jax.ShapeDtypeStruct((N_PAD, C_IN), jnp.float32),
    )(acc[0], acc[1], den)
    return out[:N_NODES]


# speedup vs baseline: 1.5498x; 1.5498x over previous
import jax
import jax.numpy as jnp
from jax import lax
from jax.experimental import pallas as pl
from jax.experimental.pallas import tpu as pltpu
from jax.experimental.pallas import tpu_sc as plsc

N_NODES = 10000
N_PAD = 10240
N_EDGES = 320000
C_IN = 128
CH = C_IN // 2
NC, NS, L = 2, 16, 16
E_TILE = N_EDGES // NS
K = 80
STEPS = E_TILE // K
NBUF = 4
CHUNK = 80
NCHUNK = N_NODES // CHUNK
RBLK = 1000
CBLK = 1024
PSHIFT = 14


def _prep_body(x_ref, w_ref, att_ref, h_ref, s_ref):
    h = lax.dot_general(x_ref[...], w_ref[...], (((1,), (1,)), ((), ())),
                        preferred_element_type=jnp.float32)
    h_ref[0] = h[:, :CH]
    h_ref[1] = h[:, CH:]
    s_ref[...] = lax.dot_general(h, att_ref[...], (((1,), (1,)), ((), ())),
                                 preferred_element_type=jnp.float32)


def _edge_body(hflat, sN, pk3d, acc_out, den_out,
               s_v, pk_v,
               rb0, rb1, rb2, rb3, cb0, cb1, cb2, cb3,
               av0, av1, av2, av3, den_v,
               gb0, gb1, gb2, gb3, accS,
               gs0, gs1, gs2, gs3, ss0, ss1, ss2, ss3):
    rowbufs = [rb0, rb1, rb2, rb3]
    colbufs = [cb0, cb1, cb2, cb3]
    alpha_vs = [av0, av1, av2, av3]
    gbufs = [gb0, gb1, gb2, gb3]
    gsems = [gs0, gs1, gs2, gs3]
    ssems = [ss0, ss1, ss2, ss3]

    c = lax.axis_index("c")
    s = lax.axis_index("s")

    pltpu.sync_copy(pk3d.at[s], pk_v)
    pltpu.sync_copy(sN, s_v)

    def drow(r, _):
        den_v[pl.ds(r * L, L)] = jnp.zeros((L,), jnp.float32)
        return 0
    lax.fori_loop(0, N_NODES // L, drow, 0)

    def zrow(r, _):
        for v in range(CH // L):
            gb0[r, pl.ds(v * L, L)] = jnp.zeros((L,), jnp.float32)
        return 0
    lax.fori_loop(0, CHUNK, zrow, 0)
    for i in range((NCHUNK + NS - 1) // NS):
        cid = i * NS + s

        @pl.when(cid < NCHUNK)
        def _():
            pltpu.sync_copy(gb0, accS.at[pl.ds(cid * CHUNK, CHUNK)])
    plsc.subcore_barrier()

    hoff = c * N_NODES

    def prep(j, b):
        for k in range(K // L):
            pk = pk_v[j, pl.ds(k * L, L)]
            rv = lax.shift_right_logical(pk, PSHIFT)
            cv = lax.bitwise_and(pk, (1 << PSHIFT) - 1)
            rowbufs[b][pl.ds(k * L, L)] = rv + hoff
            colbufs[b][pl.ds(k * L, L)] = cv
            t = (plsc.load_gather(s_v, [rv * 2])
                 + plsc.load_gather(s_v, [cv * 2 + 1]))
            t = jnp.maximum(t, t * 0.2)
            a = jnp.exp(t)
            alpha_vs[b][pl.ds(k * L, L)] = a
            plsc.addupdate_scatter(den_v, [cv], a)
        pltpu.async_copy(hflat.at[rowbufs[b]], gbufs[b], gsems[b])

    def wait_scatter(b):
        pltpu.make_async_copy(gbufs[b], accS.at[colbufs[b]],
                              ssems[b]).wait()

    def consume(j, b):
        pltpu.make_async_copy(hflat.at[rowbufs[b]], gbufs[b],
                              gsems[b]).wait()
        gb = gbufs[b]
        av = alpha_vs[b]

        def srow4(r4, _):
            r0 = r4 * 4
            for i in range(4):
                a = av[pl.ds(r0 + i, L)][0]
                for v in range(CH // L):
                    sl = pl.ds(v * L, L)
                    gb[r0 + i, sl] = gb[r0 + i, sl] * a
            return 0
        lax.fori_loop(0, K // 4, srow4, 0)
        pltpu.async_copy(gbufs[b], accS.at[colbufs[b]], ssems[b],
                         add=True)

    prep(0, 0)

    def quad(i, _):
        for b in range(NBUF):
            j = NBUF * i + b
            nb = (b + 1) % NBUF

            @pl.when(jnp.logical_and(j + 1 < STEPS, j >= NBUF - 1))
            def _():
                wait_scatter(nb)

            @pl.when(j + 1 < STEPS)
            def _():
                prep(j + 1, nb)

            @pl.when(j < STEPS)
            def _():
                consume(j, b)
        return 0

    lax.fori_loop(0, (STEPS + NBUF - 1) // NBUF, quad, 0)
    for jt in range(STEPS - NBUF, STEPS):
        wait_scatter(jt % NBUF)

    @pl.when(c == 0)
    def _():
        pltpu.sync_copy(den_v, den_out.at[s, pl.ds(0, N_NODES)])
    plsc.subcore_barrier()
    for i in range((NCHUNK + NS - 1) // NS):
        cid = i * NS + s

        @pl.when(cid < NCHUNK)
        def _():
            pltpu.sync_copy(accS.at[pl.ds(cid * CHUNK, CHUNK)],
                            acc_out.at[c, pl.ds(cid * CHUNK, CHUNK)])


def _combine_body(a0_ref, a1_ref, den_ref, o_ref):
    den = jnp.sum(den_ref[...], axis=0) + 1e-16
    o_ref[...] = jnp.concatenate([a0_ref[...], a1_ref[...]],
                                 axis=1) / den[:, None]


def kernel(x, edge_index, W, att):
    x = x.astype(jnp.float32)
    ei = edge_index.astype(jnp.int32)
    packed = (ei[0] * (1 << PSHIFT) + ei[1]).reshape(NS, STEPS, K)
    att_r = att.reshape(2, C_IN)

    hsplit, sN = pl.pallas_call(
        _prep_body,
        grid=(N_NODES // RBLK,),
        in_specs=[
            pl.BlockSpec((RBLK, C_IN), lambda i: (i, 0)),
            pl.BlockSpec((C_IN, C_IN), lambda i: (0, 0)),
            pl.BlockSpec((2, C_IN), lambda i: (0, 0)),
        ],
        out_specs=[
            pl.BlockSpec((2, RBLK, CH), lambda i: (0, i, 0)),
            pl.BlockSpec((RBLK, 2), lambda i: (i, 0)),
        ],
        out_shape=[
            jax.ShapeDtypeStruct((2, N_NODES, CH), jnp.float32),
            jax.ShapeDtypeStruct((N_NODES, 2), jnp.float32),
        ],
    )(x, W, att_r)

    mesh = plsc.VectorSubcoreMesh(core_axis_name="c", subcore_axis_name="s",
                                  num_cores=NC, num_subcores=NS)
    acc, den = pl.kernel(
        _edge_body,
        out_type=[
            jax.ShapeDtypeStruct((NC, N_PAD, CH), jnp.float32),
            jax.ShapeDtypeStruct((NS, N_PAD), jnp.float32),
        ],
        mesh=mesh,
        compiler_params=pltpu.CompilerParams(needs_layout_passes=False,
                                             use_tc_tiling_on_sc=False),
        scratch_types=[
            pltpu.VMEM((2 * N_NODES,), jnp.float32),
            pltpu.VMEM((STEPS, K), jnp.int32),
            *([pltpu.VMEM((K,), jnp.int32)] * (2 * NBUF)),
            *([pltpu.VMEM((K + L,), jnp.float32)] * NBUF),
            pltpu.VMEM((N_NODES,), jnp.float32),
            *([pltpu.VMEM((K, CH), jnp.float32)] * NBUF),
            pltpu.VMEM_SHARED((N_NODES, CH), jnp.float32),
            *([pltpu.SemaphoreType.DMA] * (2 * NBUF)),
        ],
    )(hsplit.reshape(2 * N_NODES, CH), sN.reshape(2 * N_NODES), packed)

    out = pl.pallas_call(
        _combine_body,
        grid=(N_PAD // CBLK,),
        in_specs=[
            pl.BlockSpec((CBLK, CH), lambda i: (i, 0)),
            pl.BlockSpec((CBLK, CH), lambda i: (i, 0)),
            pl.BlockSpec((NS, CBLK), lambda i: (0, i)),
        ],
        out_specs=pl.BlockSpec((CBLK, C_IN), lambda i: (i, 0)),
        out_shape=jax.ShapeDtypeStruct((N_PAD, C_IN), jnp.float32),
    )(acc[0], acc[1], den)
    return out[:N_NODES]
